# baseline (speedup 1.0000x reference)
import functools

import jax
import jax.numpy as jnp
from jax import lax
from jax.experimental import pallas as pl
from jax.experimental.pallas import tpu as pltpu
from jax.experimental.pallas import tpu_sc as plsc

DIM = 64
VOCAB = 100000
BATCH = 16384
NC = 2
NS = 16
L = 16
NW = NC * NS
BPW = BATCH // NW
CHUNK = 128
NCHUNK = BPW // CHUNK
WIDE = 2 * DIM
NH = DIM // L

HALF = 50048
TW = 2944
_GRID = HALF // TW


def _transpose_body(wlo_ref, whi_ref, vlo_ref, vhi_ref, wi2_ref, wj2_ref):
    wi2_ref[...] = jnp.concatenate(
        [wlo_ref[...].T, whi_ref[...].T], axis=1)
    wj2_ref[...] = jnp.concatenate(
        [vlo_ref[...].T, vhi_ref[...].T], axis=1)


def _relayout(wit, wjt):
    return pl.pallas_call(
        _transpose_body,
        grid=(_GRID,),
        in_specs=[
            pl.BlockSpec((DIM, TW), lambda g: (0, g)),
            pl.BlockSpec((DIM, TW), lambda g: (0, g + _GRID)),
            pl.BlockSpec((DIM, TW), lambda g: (0, g)),
            pl.BlockSpec((DIM, TW), lambda g: (0, g + _GRID)),
        ],
        out_specs=[
            pl.BlockSpec((TW, WIDE), lambda g: (g, 0)),
            pl.BlockSpec((TW, WIDE), lambda g: (g, 0)),
        ],
        out_shape=[
            jax.ShapeDtypeStruct((HALF, WIDE), jnp.float32),
            jax.ShapeDtypeStruct((HALF, WIDE), jnp.float32),
        ],
    )(wit, whi := wit, wjt, wjt)


_mesh = plsc.VectorSubcoreMesh(core_axis_name="c", subcore_axis_name="s")


@functools.partial(
    pl.kernel,
    out_type=jax.ShapeDtypeStruct((BATCH,), jnp.float32),
    mesh=_mesh,
    compiler_params=pltpu.CompilerParams(
        needs_layout_passes=False, use_tc_tiling_on_sc=True),
    scratch_types=[
        pltpu.VMEM((BPW,), jnp.int32),
        pltpu.VMEM((BPW,), jnp.int32),
        pltpu.VMEM((BPW,), jnp.int32),
        pltpu.VMEM((BPW,), jnp.int32),
        pltpu.VMEM((BPW,), jnp.int32),
        pltpu.VMEM((BPW,), jnp.int32),
        pltpu.VMEM((CHUNK, WIDE), jnp.float32),
        pltpu.VMEM((CHUNK, WIDE), jnp.float32),
        pltpu.VMEM((CHUNK, WIDE), jnp.float32),
        pltpu.VMEM((CHUNK, WIDE), jnp.float32),
        pltpu.VMEM((BPW,), jnp.float32),
        pltpu.VMEM((BPW,), jnp.float32),
        pltpu.VMEM((BPW,), jnp.float32),
        pltpu.SemaphoreType.DMA,
        pltpu.SemaphoreType.DMA,
        pltpu.SemaphoreType.DMA,
    ],
)
def _glove_sc(i_idx, j_idx, wi2, wj2, bi_flat, bj_flat, out_hbm,
              idx_i, idx_j, ihalf, jhalf, coloff_i, coloff_j,
              ri0, ri1, rj0, rj1, bias_i, bias_j, out_v,
              sem0, sem1, sem_b):
    wid = lax.axis_index("s") * NC + lax.axis_index("c")
    base = wid * BPW
    ri = (ri0, ri1)
    rj = (rj0, rj1)
    sems = (sem0, sem1)

    pltpu.sync_copy(i_idx.at[pl.ds(base, BPW)], idx_i)
    pltpu.sync_copy(j_idx.at[pl.ds(base, BPW)], idx_j)

    bias_h = [
        pltpu.async_copy(bi_flat.at[idx_i], bias_i, sem_b),
        pltpu.async_copy(bj_flat.at[idx_j], bias_j, sem_b),
    ]

    def split_body(t, carry):
        o = pl.ds(pl.multiple_of(t * L, L), L)
        ii = idx_i[o]
        jj = idx_j[o]
        hi_i = jnp.where(ii >= HALF, 1, 0)
        hi_j = jnp.where(jj >= HALF, 1, 0)
        ihalf[o] = ii - hi_i * HALF
        jhalf[o] = jj - hi_j * HALF
        coloff_i[o] = hi_i * DIM
        coloff_j[o] = hi_j * DIM
        return carry

    lax.fori_loop(0, BPW // L, split_body, 0)

    def fire(c):
        b = c % 2
        rows = pl.ds(c * CHUNK, CHUNK)
        return (pltpu.async_copy(wi2.at[ihalf.at[rows]], ri[b], sems[b]),
                pltpu.async_copy(wj2.at[jhalf.at[rows]], rj[b], sems[b]))

    handles = {0: fire(0), 1: fire(1)}
    for h in bias_h:
        h.wait()

    iota = lax.iota(jnp.int32, L)

    for c in range(NCHUNK):
        b = c % 2
        for h in handles.pop(c):
            h.wait()

        a_ref = ri[b]
        b_ref = rj[b]

        def body(g, carry):
            gb = pl.ds(c * CHUNK + g * L, L)
            row16 = g * L + iota
            ci = coloff_i[gb]
            cj = coloff_j[gb]
            acc = bias_i[gb] + bias_j[gb]
            for d in range(DIM):
                acc += (plsc.load_gather(a_ref, [row16, ci + d])
                        * plsc.load_gather(b_ref, [row16, cj + d]))
            out_v[gb] = acc
            return carry

        lax.fori_loop(0, CHUNK // L, body, 0)
        if c + 2 < NCHUNK:
            handles[c + 2] = fire(c + 2)

    pltpu.sync_copy(out_v, out_hbm.at[pl.ds(base, BPW)])


def kernel(i_idx, j_idx, wi, wj, bi, bj):
    wi2, wj2 = _relayout(wi.T, wj.T)
    return _glove_sc(i_idx.astype(jnp.int32), j_idx.astype(jnp.int32),
                     wi2, wj2, bi.T.reshape(-1), bj.T.reshape(-1))


# speedup vs baseline: 1.0825x; 1.0825x over previous
import functools

import jax
import jax.numpy as jnp
from jax import lax
from jax.experimental import pallas as pl
from jax.experimental.pallas import tpu as pltpu
from jax.experimental.pallas import tpu_sc as plsc

DIM = 64
VOCAB = 100000
BATCH = 16384
NC = 2
NS = 16
L = 16
NW = NC * NS
BPW = BATCH // NW
CHUNK = 128
NCHUNK = BPW // CHUNK
WIDE = 2 * DIM
NH = DIM // L

HALF = 50048
TW = 2944
_GRID = HALF // TW


def _transpose_body(wlo_ref, whi_ref, vlo_ref, vhi_ref, wi2_ref, wj2_ref):
    wi2_ref[...] = jnp.concatenate(
        [wlo_ref[...].T, whi_ref[...].T], axis=1)
    wj2_ref[...] = jnp.concatenate(
        [vlo_ref[...].T, vhi_ref[...].T], axis=1)


def _relayout(wit, wjt):
    return pl.pallas_call(
        _transpose_body,
        grid=(_GRID,),
        in_specs=[
            pl.BlockSpec((DIM, TW), lambda g: (0, g)),
            pl.BlockSpec((DIM, TW), lambda g: (0, g + _GRID)),
            pl.BlockSpec((DIM, TW), lambda g: (0, g)),
            pl.BlockSpec((DIM, TW), lambda g: (0, g + _GRID)),
        ],
        out_specs=[
            pl.BlockSpec((TW, WIDE), lambda g: (g, 0)),
            pl.BlockSpec((TW, WIDE), lambda g: (g, 0)),
        ],
        out_shape=[
            jax.ShapeDtypeStruct((HALF, WIDE), jnp.float32),
            jax.ShapeDtypeStruct((HALF, WIDE), jnp.float32),
        ],
    )(wit, whi := wit, wjt, wjt)


_mesh = plsc.VectorSubcoreMesh(core_axis_name="c", subcore_axis_name="s")


@functools.partial(
    pl.kernel,
    out_type=jax.ShapeDtypeStruct((BATCH,), jnp.float32),
    mesh=_mesh,
    compiler_params=pltpu.CompilerParams(
        needs_layout_passes=False, use_tc_tiling_on_sc=True),
    scratch_types=[
        pltpu.VMEM((BPW,), jnp.int32),
        pltpu.VMEM((BPW,), jnp.int32),
        pltpu.VMEM((BPW,), jnp.int32),
        pltpu.VMEM((BPW,), jnp.int32),
        pltpu.VMEM((BPW,), jnp.int32),
        pltpu.VMEM((BPW,), jnp.int32),
        pltpu.VMEM((CHUNK, WIDE), jnp.float32),
        pltpu.VMEM((CHUNK, WIDE), jnp.float32),
        pltpu.VMEM((CHUNK, WIDE), jnp.float32),
        pltpu.VMEM((CHUNK, WIDE), jnp.float32),
        pltpu.VMEM((BPW,), jnp.float32),
        pltpu.VMEM((BPW,), jnp.float32),
        pltpu.VMEM((BPW,), jnp.float32),
        pltpu.SemaphoreType.DMA,
        pltpu.SemaphoreType.DMA,
        pltpu.SemaphoreType.DMA,
    ],
)
def _glove_sc(i_idx, j_idx, wi2, wj2, bi_flat, bj_flat, out_hbm,
              idx_i, idx_j, ihalf, jhalf, coloff_i, coloff_j,
              ri0, ri1, rj0, rj1, bias_i, bias_j, out_v,
              sem0, sem1, sem_b):
    wid = lax.axis_index("s") * NC + lax.axis_index("c")
    base = wid * BPW
    ri = (ri0, ri1)
    rj = (rj0, rj1)
    sems = (sem0, sem1)

    pltpu.sync_copy(i_idx.at[pl.ds(base, BPW)], idx_i)
    pltpu.sync_copy(j_idx.at[pl.ds(base, BPW)], idx_j)

    bias_h = [
        pltpu.async_copy(bi_flat.at[idx_i], bias_i, sem_b),
        pltpu.async_copy(bj_flat.at[idx_j], bias_j, sem_b),
    ]

    def split_body(t, carry):
        o = pl.ds(pl.multiple_of(t * L, L), L)
        ii = idx_i[o]
        jj = idx_j[o]
        hi_i = jnp.where(ii >= HALF, 1, 0)
        hi_j = jnp.where(jj >= HALF, 1, 0)
        ihalf[o] = ii - hi_i * HALF
        jhalf[o] = jj - hi_j * HALF
        coloff_i[o] = hi_i * DIM
        coloff_j[o] = hi_j * DIM
        return carry

    lax.fori_loop(0, BPW // L, split_body, 0)

    def fire(c):
        b = c % 2
        rows = pl.ds(c * CHUNK, CHUNK)
        return (pltpu.async_copy(wi2.at[ihalf.at[rows]], ri[b], sems[b]),
                pltpu.async_copy(wj2.at[jhalf.at[rows]], rj[b], sems[b]))

    handles = {0: fire(0), 1: fire(1)}
    for h in bias_h:
        h.wait()

    iota = lax.iota(jnp.int32, L)

    for c in range(NCHUNK):
        b = c % 2
        for h in handles.pop(c):
            h.wait()

        a_ref = ri[b]
        b_ref = rj[b]

        def body(g, carry):
            gb = pl.ds(c * CHUNK + g * L, L)
            row16 = g * L + iota
            ci = coloff_i[gb]
            cj = coloff_j[gb]
            acc0 = bias_i[gb] + bias_j[gb]

            def dblk(db, accv):
                cib = ci + db * L
                cjb = cj + db * L
                for dd in range(L):
                    accv += (plsc.load_gather(a_ref, [row16, cib + dd])
                             * plsc.load_gather(b_ref, [row16, cjb + dd]))
                return accv

            out_v[gb] = lax.fori_loop(0, DIM // L, dblk, acc0)
            return carry

        lax.fori_loop(0, CHUNK // L, body, 0)
        if c + 2 < NCHUNK:
            handles[c + 2] = fire(c + 2)

    pltpu.sync_copy(out_v, out_hbm.at[pl.ds(base, BPW)])


def kernel(i_idx, j_idx, wi, wj, bi, bj):
    wi2, wj2 = _relayout(wi.T, wj.T)
    return _glove_sc(i_idx.astype(jnp.int32), j_idx.astype(jnp.int32),
                     wi2, wj2, bi.T.reshape(-1), bj.T.reshape(-1))


# speedup vs baseline: 1.2811x; 1.1834x over previous
import functools

import jax
import jax.numpy as jnp
from jax import lax
from jax.experimental import pallas as pl
from jax.experimental.pallas import tpu as pltpu
from jax.experimental.pallas import tpu_sc as plsc

DIM = 64
VOCAB = 100000
BATCH = 16384
NC = 2
NS = 16
L = 16
NW = NC * NS
BPW = BATCH // NW
CHUNK = 256
NCHUNK = BPW // CHUNK
WIDE = 2 * DIM
NH = DIM // L

HALF = 50048
TW = 2944
_GRID = HALF // TW


def _transpose_body(wlo_ref, whi_ref, vlo_ref, vhi_ref, wi2_ref, wj2_ref):
    wi2_ref[...] = jnp.concatenate(
        [wlo_ref[...].T, whi_ref[...].T], axis=1)
    wj2_ref[...] = jnp.concatenate(
        [vlo_ref[...].T, vhi_ref[...].T], axis=1)


def _relayout(wit, wjt):
    return pl.pallas_call(
        _transpose_body,
        grid=(_GRID,),
        in_specs=[
            pl.BlockSpec((DIM, TW), lambda g: (0, g)),
            pl.BlockSpec((DIM, TW), lambda g: (0, g + _GRID)),
            pl.BlockSpec((DIM, TW), lambda g: (0, g)),
            pl.BlockSpec((DIM, TW), lambda g: (0, g + _GRID)),
        ],
        out_specs=[
            pl.BlockSpec((TW, WIDE), lambda g: (g, 0)),
            pl.BlockSpec((TW, WIDE), lambda g: (g, 0)),
        ],
        out_shape=[
            jax.ShapeDtypeStruct((HALF, WIDE), jnp.float32),
            jax.ShapeDtypeStruct((HALF, WIDE), jnp.float32),
        ],
    )(wit, whi := wit, wjt, wjt)


_mesh = plsc.VectorSubcoreMesh(core_axis_name="c", subcore_axis_name="s")


@functools.partial(
    pl.kernel,
    out_type=jax.ShapeDtypeStruct((BATCH,), jnp.float32),
    mesh=_mesh,
    compiler_params=pltpu.CompilerParams(
        needs_layout_passes=False, use_tc_tiling_on_sc=True),
    scratch_types=[
        pltpu.VMEM((BPW,), jnp.int32),
        pltpu.VMEM((BPW,), jnp.int32),
        pltpu.VMEM((BPW,), jnp.int32),
        pltpu.VMEM((BPW,), jnp.int32),
        pltpu.VMEM((BPW,), jnp.int32),
        pltpu.VMEM((CHUNK, WIDE), jnp.float32),
        pltpu.VMEM((CHUNK, WIDE), jnp.float32),
        pltpu.VMEM((BPW,), jnp.float32),
        pltpu.VMEM((BPW,), jnp.float32),
        pltpu.VMEM((CHUNK * L,), jnp.float32),
        pltpu.VMEM((CHUNK * L,), jnp.float32),
        pltpu.VMEM((CHUNK * L,), jnp.float32),
        pltpu.VMEM((CHUNK * L,), jnp.float32),
        pltpu.VMEM((BPW,), jnp.float32),
        pltpu.SemaphoreType.DMA,
        pltpu.SemaphoreType.DMA,
    ],
)
def _glove_sc(i_idx, j_idx, wi2, wj2, bi_flat, bj_flat, out_hbm,
              idx_i, idx_j, ihalf, jhalf, sel, rows_i, rows_j,
              bias_i, bias_j, p00, p01, p10, p11, out_v, sem, sem_b):
    wid = lax.axis_index("s") * NC + lax.axis_index("c")
    base = wid * BPW

    pltpu.sync_copy(i_idx.at[pl.ds(base, BPW)], idx_i)
    pltpu.sync_copy(j_idx.at[pl.ds(base, BPW)], idx_j)

    bias_h = [
        pltpu.async_copy(bi_flat.at[idx_i], bias_i, sem_b),
        pltpu.async_copy(bj_flat.at[idx_j], bias_j, sem_b),
    ]

    def split_body(t, carry):
        o = pl.ds(pl.multiple_of(t * L, L), L)
        ii = idx_i[o]
        jj = idx_j[o]
        hi_i = jnp.where(ii >= HALF, 1, 0)
        hi_j = jnp.where(jj >= HALF, 1, 0)
        ihalf[o] = ii - hi_i * HALF
        jhalf[o] = jj - hi_j * HALF
        sel[o] = hi_i * 2 + hi_j
        return carry

    lax.fori_loop(0, BPW // L, split_body, 0)
    for h in bias_h:
        h.wait()

    iota = lax.iota(jnp.int32, L)

    def chunk_body(c, carry):
        rows = pl.ds(c * CHUNK, CHUNK)
        handles = [
            pltpu.async_copy(wi2.at[ihalf.at[rows]], rows_i, sem),
            pltpu.async_copy(wj2.at[jhalf.at[rows]], rows_j, sem),
        ]
        for h in handles:
            h.wait()

        def body1(t, carry1):
            av = [rows_i[t, pl.ds(k * L, L)] for k in range(WIDE // L)]
            bv = [rows_j[t, pl.ds(k * L, L)] for k in range(WIDE // L)]
            m00 = av[0] * bv[0]
            m01 = av[0] * bv[NH]
            m10 = av[NH] * bv[0]
            m11 = av[NH] * bv[NH]
            for k in range(1, NH):
                m00 += av[k] * bv[k]
                m01 += av[k] * bv[NH + k]
                m10 += av[NH + k] * bv[k]
                m11 += av[NH + k] * bv[NH + k]
            o = pl.ds(pl.multiple_of(t * L, L), L)
            p00[o] = m00
            p01[o] = m01
            p10[o] = m10
            p11[o] = m11
            return carry1

        lax.fori_loop(0, CHUNK, body1, 0)

        def body2(g, carry2):
            flat = g * (L * L) + iota * L
            a00 = plsc.load_gather(p00, [flat])
            a01 = plsc.load_gather(p01, [flat])
            a10 = plsc.load_gather(p10, [flat])
            a11 = plsc.load_gather(p11, [flat])
            for k in range(1, L):
                a00 += plsc.load_gather(p00, [flat + k])
                a01 += plsc.load_gather(p01, [flat + k])
                a10 += plsc.load_gather(p10, [flat + k])
                a11 += plsc.load_gather(p11, [flat + k])
            o = pl.ds(c * CHUNK + g * L, L)
            sv = sel[o]
            res = jnp.where(sv == 0, a00,
                            jnp.where(sv == 1, a01,
                                      jnp.where(sv == 2, a10, a11)))
            out_v[o] = res + bias_i[o] + bias_j[o]
            return carry2

        lax.fori_loop(0, CHUNK // L, body2, 0)
        return carry

    lax.fori_loop(0, NCHUNK, chunk_body, 0)

    pltpu.sync_copy(out_v, out_hbm.at[pl.ds(base, BPW)])


def kernel(i_idx, j_idx, wi, wj, bi, bj):
    wi2, wj2 = _relayout(wi.T, wj.T)
    return _glove_sc(i_idx.astype(jnp.int32), j_idx.astype(jnp.int32),
                     wi2, wj2, bi.T.reshape(-1), bj.T.reshape(-1))


# speedup vs baseline: 1.3204x; 1.0307x over previous
import functools

import jax
import jax.numpy as jnp
from jax import lax
from jax.experimental import pallas as pl
from jax.experimental.pallas import tpu as pltpu
from jax.experimental.pallas import tpu_sc as plsc

DIM = 64
VOCAB = 100000
BATCH = 16384
NC = 2
NS = 16
L = 16
NW = NC * NS
BPW = BATCH // NW
CHUNK = 128
NCHUNK = BPW // CHUNK
WIDE = 2 * DIM
NH = DIM // L

HALF = 50048
TW = 2944
_GRID = HALF // TW


def _transpose_body(wlo_ref, whi_ref, vlo_ref, vhi_ref, wi2_ref, wj2_ref):
    wi2_ref[...] = jnp.concatenate(
        [wlo_ref[...].T, whi_ref[...].T], axis=1)
    wj2_ref[...] = jnp.concatenate(
        [vlo_ref[...].T, vhi_ref[...].T], axis=1)


def _relayout(wit, wjt):
    return pl.pallas_call(
        _transpose_body,
        grid=(_GRID,),
        in_specs=[
            pl.BlockSpec((DIM, TW), lambda g: (0, g)),
            pl.BlockSpec((DIM, TW), lambda g: (0, g + _GRID)),
            pl.BlockSpec((DIM, TW), lambda g: (0, g)),
            pl.BlockSpec((DIM, TW), lambda g: (0, g + _GRID)),
        ],
        out_specs=[
            pl.BlockSpec((TW, WIDE), lambda g: (g, 0)),
            pl.BlockSpec((TW, WIDE), lambda g: (g, 0)),
        ],
        out_shape=[
            jax.ShapeDtypeStruct((HALF, WIDE), jnp.float32),
            jax.ShapeDtypeStruct((HALF, WIDE), jnp.float32),
        ],
    )(wit, whi := wit, wjt, wjt)


_mesh = plsc.VectorSubcoreMesh(core_axis_name="c", subcore_axis_name="s")


@functools.partial(
    pl.kernel,
    out_type=jax.ShapeDtypeStruct((BATCH,), jnp.float32),
    mesh=_mesh,
    compiler_params=pltpu.CompilerParams(
        needs_layout_passes=False, use_tc_tiling_on_sc=True),
    scratch_types=[
        pltpu.VMEM((BPW,), jnp.int32),
        pltpu.VMEM((BPW,), jnp.int32),
        pltpu.VMEM((BPW,), jnp.int32),
        pltpu.VMEM((BPW,), jnp.int32),
        pltpu.VMEM((BPW,), jnp.int32),
        pltpu.VMEM((CHUNK, WIDE), jnp.float32),
        pltpu.VMEM((CHUNK, WIDE), jnp.float32),
        pltpu.VMEM((CHUNK, WIDE), jnp.float32),
        pltpu.VMEM((CHUNK, WIDE), jnp.float32),
        pltpu.VMEM((BPW,), jnp.float32),
        pltpu.VMEM((BPW,), jnp.float32),
        pltpu.VMEM((CHUNK * L,), jnp.float32),
        pltpu.VMEM((CHUNK * L,), jnp.float32),
        pltpu.VMEM((CHUNK * L,), jnp.float32),
        pltpu.VMEM((CHUNK * L,), jnp.float32),
        pltpu.VMEM((BPW,), jnp.float32),
        pltpu.SemaphoreType.DMA,
        pltpu.SemaphoreType.DMA,
        pltpu.SemaphoreType.DMA,
    ],
)
def _glove_sc(i_idx, j_idx, wi2, wj2, bi_flat, bj_flat, out_hbm,
              idx_i, idx_j, ihalf, jhalf, sel, ri0, ri1, rj0, rj1,
              bias_i, bias_j, p00, p01, p10, p11, out_v,
              sem0, sem1, sem_b):
    ri = (ri0, ri1)
    rj = (rj0, rj1)
    sems = (sem0, sem1)
    wid = lax.axis_index("s") * NC + lax.axis_index("c")
    base = wid * BPW

    pltpu.sync_copy(i_idx.at[pl.ds(base, BPW)], idx_i)
    pltpu.sync_copy(j_idx.at[pl.ds(base, BPW)], idx_j)

    bias_h = [
        pltpu.async_copy(bi_flat.at[idx_i], bias_i, sem_b),
        pltpu.async_copy(bj_flat.at[idx_j], bias_j, sem_b),
    ]

    def split_body(t, carry):
        o = pl.ds(pl.multiple_of(t * L, L), L)
        ii = idx_i[o]
        jj = idx_j[o]
        hi_i = jnp.where(ii >= HALF, 1, 0)
        hi_j = jnp.where(jj >= HALF, 1, 0)
        ihalf[o] = ii - hi_i * HALF
        jhalf[o] = jj - hi_j * HALF
        sel[o] = hi_i * 2 + hi_j
        return carry

    lax.fori_loop(0, BPW // L, split_body, 0)
    for h in bias_h:
        h.wait()

    iota = lax.iota(jnp.int32, L)

    def fire(c):
        b = c % 2
        rows = pl.ds(c * CHUNK, CHUNK)
        return (pltpu.async_copy(wi2.at[ihalf.at[rows]], ri[b], sems[b]),
                pltpu.async_copy(wj2.at[jhalf.at[rows]], rj[b], sems[b]))

    inflight = {0: fire(0), 1: fire(1)}

    for c in range(NCHUNK):
        b = c % 2
        rows_i = ri[b]
        rows_j = rj[b]
        for h in inflight.pop(c):
            h.wait()

        def body1(t, carry1):
            av = [rows_i[t, pl.ds(k * L, L)] for k in range(WIDE // L)]
            bv = [rows_j[t, pl.ds(k * L, L)] for k in range(WIDE // L)]
            m00 = av[0] * bv[0]
            m01 = av[0] * bv[NH]
            m10 = av[NH] * bv[0]
            m11 = av[NH] * bv[NH]
            for k in range(1, NH):
                m00 += av[k] * bv[k]
                m01 += av[k] * bv[NH + k]
                m10 += av[NH + k] * bv[k]
                m11 += av[NH + k] * bv[NH + k]
            o = pl.ds(pl.multiple_of(t * L, L), L)
            p00[o] = m00
            p01[o] = m01
            p10[o] = m10
            p11[o] = m11
            return carry1

        lax.fori_loop(0, CHUNK, body1, 0)

        def body2(g, carry2):
            flat = g * (L * L) + iota * L
            a00 = plsc.load_gather(p00, [flat])
            a01 = plsc.load_gather(p01, [flat])
            a10 = plsc.load_gather(p10, [flat])
            a11 = plsc.load_gather(p11, [flat])
            for k in range(1, L):
                a00 += plsc.load_gather(p00, [flat + k])
                a01 += plsc.load_gather(p01, [flat + k])
                a10 += plsc.load_gather(p10, [flat + k])
                a11 += plsc.load_gather(p11, [flat + k])
            o = pl.ds(c * CHUNK + g * L, L)
            sv = sel[o]
            res = jnp.where(sv == 0, a00,
                            jnp.where(sv == 1, a01,
                                      jnp.where(sv == 2, a10, a11)))
            out_v[o] = res + bias_i[o] + bias_j[o]
            return carry2

        lax.fori_loop(0, CHUNK // L, body2, 0)
        if c + 2 < NCHUNK:
            inflight[c + 2] = fire(c + 2)

    pltpu.sync_copy(out_v, out_hbm.at[pl.ds(base, BPW)])


def kernel(i_idx, j_idx, wi, wj, bi, bj):
    wi2, wj2 = _relayout(wi.T, wj.T)
    return _glove_sc(i_idx.astype(jnp.int32), j_idx.astype(jnp.int32),
                     wi2, wj2, bi.T.reshape(-1), bj.T.reshape(-1))


# speedup vs baseline: 1.3720x; 1.0390x over previous
import functools

import jax
import jax.numpy as jnp
from jax import lax
from jax.experimental import pallas as pl
from jax.experimental.pallas import tpu as pltpu
from jax.experimental.pallas import tpu_sc as plsc

DIM = 64
VOCAB = 100000
BATCH = 16384
NC = 2
NS = 16
L = 16
NW = NC * NS
BPW = BATCH // NW
CHUNK = 128
NCHUNK = BPW // CHUNK
WIDE = 2 * DIM
NH = DIM // L

HALF = 50048
TW = 2944
_GRID = HALF // TW


def _transpose_body(wlo_ref, whi_ref, vlo_ref, vhi_ref, wi2_ref, wj2_ref):
    wi2_ref[...] = jnp.concatenate(
        [wlo_ref[...].T, whi_ref[...].T], axis=1)
    wj2_ref[...] = jnp.concatenate(
        [vlo_ref[...].T, vhi_ref[...].T], axis=1)


def _relayout(wit, wjt):
    return pl.pallas_call(
        _transpose_body,
        grid=(_GRID,),
        in_specs=[
            pl.BlockSpec((DIM, TW), lambda g: (0, g)),
            pl.BlockSpec((DIM, TW), lambda g: (0, g + _GRID)),
            pl.BlockSpec((DIM, TW), lambda g: (0, g)),
            pl.BlockSpec((DIM, TW), lambda g: (0, g + _GRID)),
        ],
        out_specs=[
            pl.BlockSpec((TW, WIDE), lambda g: (g, 0)),
            pl.BlockSpec((TW, WIDE), lambda g: (g, 0)),
        ],
        out_shape=[
            jax.ShapeDtypeStruct((HALF, WIDE), jnp.float32),
            jax.ShapeDtypeStruct((HALF, WIDE), jnp.float32),
        ],
    )(wit, whi := wit, wjt, wjt)


_mesh = plsc.VectorSubcoreMesh(core_axis_name="c", subcore_axis_name="s")


@functools.partial(
    pl.kernel,
    out_type=jax.ShapeDtypeStruct((BATCH,), jnp.float32),
    mesh=_mesh,
    compiler_params=pltpu.CompilerParams(
        needs_layout_passes=False, use_tc_tiling_on_sc=True),
    scratch_types=[
        pltpu.VMEM((BPW,), jnp.int32),
        pltpu.VMEM((BPW,), jnp.int32),
        pltpu.VMEM((BPW,), jnp.int32),
        pltpu.VMEM((BPW,), jnp.int32),
        pltpu.VMEM((BPW,), jnp.int32),
        pltpu.VMEM((CHUNK, WIDE), jnp.float32),
        pltpu.VMEM((CHUNK, WIDE), jnp.float32),
        pltpu.VMEM((CHUNK, WIDE), jnp.float32),
        pltpu.VMEM((CHUNK, WIDE), jnp.float32),
        pltpu.VMEM((BPW,), jnp.float32),
        pltpu.VMEM((BPW,), jnp.float32),
        pltpu.VMEM((4 * CHUNK * L,), jnp.float32),
        pltpu.VMEM((BPW,), jnp.float32),
        pltpu.SemaphoreType.DMA,
        pltpu.SemaphoreType.DMA,
        pltpu.SemaphoreType.DMA,
    ],
)
def _glove_sc(i_idx, j_idx, wi2, wj2, bi_flat, bj_flat, out_hbm,
              idx_i, idx_j, ihalf, jhalf, sel, ri0, ri1, rj0, rj1,
              bias_i, bias_j, pbuf, out_v,
              sem0, sem1, sem_b):
    ri = (ri0, ri1)
    rj = (rj0, rj1)
    sems = (sem0, sem1)
    wid = lax.axis_index("s") * NC + lax.axis_index("c")
    base = wid * BPW

    pltpu.sync_copy(i_idx.at[pl.ds(base, BPW)], idx_i)
    pltpu.sync_copy(j_idx.at[pl.ds(base, BPW)], idx_j)

    bias_h = [
        pltpu.async_copy(bi_flat.at[idx_i], bias_i, sem_b),
        pltpu.async_copy(bj_flat.at[idx_j], bias_j, sem_b),
    ]

    def split_body(t, carry):
        o = pl.ds(pl.multiple_of(t * L, L), L)
        ii = idx_i[o]
        jj = idx_j[o]
        hi_i = jnp.where(ii >= HALF, 1, 0)
        hi_j = jnp.where(jj >= HALF, 1, 0)
        ihalf[o] = ii - hi_i * HALF
        jhalf[o] = jj - hi_j * HALF
        sel[o] = hi_i * 2 + hi_j
        return carry

    lax.fori_loop(0, BPW // L, split_body, 0)
    for h in bias_h:
        h.wait()

    iota = lax.iota(jnp.int32, L)

    def fire(c):
        b = c % 2
        rows = pl.ds(c * CHUNK, CHUNK)
        return (pltpu.async_copy(wi2.at[ihalf.at[rows]], ri[b], sems[b]),
                pltpu.async_copy(wj2.at[jhalf.at[rows]], rj[b], sems[b]))

    inflight = {0: fire(0), 1: fire(1)}

    for c in range(NCHUNK):
        b = c % 2
        rows_i = ri[b]
        rows_j = rj[b]
        for h in inflight.pop(c):
            h.wait()

        def body1(t, carry1):
            av = [rows_i[t, pl.ds(k * L, L)] for k in range(WIDE // L)]
            bv = [rows_j[t, pl.ds(k * L, L)] for k in range(WIDE // L)]
            m00 = av[0] * bv[0]
            m01 = av[0] * bv[NH]
            m10 = av[NH] * bv[0]
            m11 = av[NH] * bv[NH]
            for k in range(1, NH):
                m00 += av[k] * bv[k]
                m01 += av[k] * bv[NH + k]
                m10 += av[NH + k] * bv[k]
                m11 += av[NH + k] * bv[NH + k]
            tl = pl.multiple_of(t * L, L)
            pbuf[pl.ds(tl, L)] = m00
            pbuf[pl.ds(tl + CHUNK * L, L)] = m01
            pbuf[pl.ds(tl + 2 * CHUNK * L, L)] = m10
            pbuf[pl.ds(tl + 3 * CHUNK * L, L)] = m11
            return carry1

        lax.fori_loop(0, CHUNK, body1, 0)

        def body2(g, carry2):
            o = pl.ds(c * CHUNK + g * L, L)
            sv = sel[o]
            flat = sv * (CHUNK * L) + g * (L * L) + iota * L
            res = plsc.load_gather(pbuf, [flat])
            for k in range(1, L):
                res += plsc.load_gather(pbuf, [flat + k])
            out_v[o] = res + bias_i[o] + bias_j[o]
            return carry2

        lax.fori_loop(0, CHUNK // L, body2, 0)
        if c + 2 < NCHUNK:
            inflight[c + 2] = fire(c + 2)

    pltpu.sync_copy(out_v, out_hbm.at[pl.ds(base, BPW)])


def kernel(i_idx, j_idx, wi, wj, bi, bj):
    wi2, wj2 = _relayout(wi.T, wj.T)
    return _glove_sc(i_idx.astype(jnp.int32), j_idx.astype(jnp.int32),
                     wi2, wj2, bi.T.reshape(-1), bj.T.reshape(-1))
